# baseline (speedup 1.0000x reference)
import functools

import jax
import jax.numpy as jnp
from jax import lax
from jax.experimental import pallas as pl
from jax.experimental.pallas import tpu as pltpu
from jax.experimental.pallas import tpu_sc as plsc

NC = 2
NS = 16
NW = NC * NS
L = 16

ROWS, COLS = 32768, 1024
TOTAL = ROWS * COLS
ELEMS_PER_W = TOTAL // NW
CHUNK_ELEMS = 32 * COLS
NCHUNK = ELEMS_PER_W // CHUNK_ELEMS
NBUF = 2
U = 8


def _chunk_max(buf):
    init = tuple(buf[pl.ds(j * L, L)] for j in range(U))

    def body(i, accs):
        base = i * (U * L)
        return tuple(
            jnp.maximum(accs[j], buf[pl.ds(base + j * L, L)]) for j in range(U)
        )

    accs = lax.fori_loop(1, CHUNK_ELEMS // (U * L), body, init)
    accs = list(accs)
    while len(accs) > 1:
        accs = [jnp.maximum(accs[i], accs[i + 1]) for i in range(0, len(accs), 2)]
    return accs[0]


_sc_mesh = plsc.VectorSubcoreMesh(core_axis_name="c", subcore_axis_name="s")


@functools.partial(
    pl.kernel,
    mesh=_sc_mesh,
    out_type=jax.ShapeDtypeStruct((NW, L), jnp.float32),
    scratch_types=[
        pltpu.VMEM((NBUF, CHUNK_ELEMS), jnp.float32),
        pltpu.VMEM((L,), jnp.float32),
        pltpu.SemaphoreType.DMA,
        pltpu.SemaphoreType.DMA,
    ],
)
def _sc_partial_max(vals_hbm, out_hbm, buf, outv, sem0, sem1):
    wid = lax.axis_index("s") * NC + lax.axis_index("c")
    base = wid * ELEMS_PER_W
    sems = (sem0, sem1)

    def start_copy(g):
        b = g % NBUF
        cp = pltpu.make_async_copy(
            vals_hbm.at[pl.ds(base + g * CHUNK_ELEMS, CHUNK_ELEMS)],
            buf.at[b],
            sems[b],
        )
        cp.start()
        return cp

    cps = [start_copy(g) for g in range(NBUF)]
    running = None
    for g in range(NCHUNK):
        cps[g].wait()
        m = _chunk_max(buf.at[g % NBUF])
        running = m if running is None else jnp.maximum(running, m)
        if g + NBUF < NCHUNK:
            cps.append(start_copy(g + NBUF))
    outv[...] = running
    pltpu.sync_copy(outv, out_hbm.at[wid])


def _combine_kernel(parts_ref, o_ref):
    o_ref[0, 0] = jnp.max(parts_ref[...])


def kernel(values, prefix_sum):
    del prefix_sum
    parts = _sc_partial_max(values.reshape(-1))
    combined = pl.pallas_call(
        _combine_kernel,
        out_shape=jax.ShapeDtypeStruct((1, 1), jnp.float32),
        out_specs=pl.BlockSpec(memory_space=pltpu.SMEM),
    )(parts)
    return combined[0, 0]


# speedup vs baseline: 1.2129x; 1.2129x over previous
import functools

import jax
import jax.numpy as jnp
from jax import lax
from jax.experimental import pallas as pl
from jax.experimental.pallas import tpu as pltpu
from jax.experimental.pallas import tpu_sc as plsc

NC = 2
NS = 16
NW = NC * NS
L = 16

ROWS, COLS = 32768, 1024
TOTAL = ROWS * COLS
ELEMS_PER_W = TOTAL // NW
CHUNK_ELEMS = 32 * COLS
NCHUNK = ELEMS_PER_W // CHUNK_ELEMS
NBUF = 3
U = 8
CHUNK_VECS = CHUNK_ELEMS // L


def _chunk_max(buf):
    init = tuple(buf[pl.ds(j * L, L)] for j in range(U))

    @plsc.parallel_loop(U, CHUNK_VECS, step=U, unroll=2, carry=init)
    def accs(i, accs):
        base = i * L
        return tuple(
            jnp.maximum(accs[j], buf[pl.ds(base + j * L, L)]) for j in range(U)
        )

    accs = list(accs)
    while len(accs) > 1:
        accs = [jnp.maximum(accs[i], accs[i + 1]) for i in range(0, len(accs), 2)]
    return accs[0]


_sc_mesh = plsc.VectorSubcoreMesh(core_axis_name="c", subcore_axis_name="s")


@functools.partial(
    pl.kernel,
    mesh=_sc_mesh,
    out_type=jax.ShapeDtypeStruct((NW, L), jnp.float32),
    scratch_types=[
        pltpu.VMEM((CHUNK_ELEMS,), jnp.float32),
        pltpu.VMEM((CHUNK_ELEMS,), jnp.float32),
        pltpu.VMEM((CHUNK_ELEMS,), jnp.float32),
        pltpu.VMEM((L,), jnp.float32),
        pltpu.SemaphoreType.DMA,
        pltpu.SemaphoreType.DMA,
        pltpu.SemaphoreType.DMA,
    ],
)
def _sc_partial_max(vals_hbm, out_hbm, buf0, buf1, buf2, outv, sem0, sem1, sem2):
    wid = lax.axis_index("s") * NC + lax.axis_index("c")
    base = wid * ELEMS_PER_W
    bufs = (buf0, buf1, buf2)
    sems = (sem0, sem1, sem2)

    def start_copy(g):
        b = g % NBUF
        cp = pltpu.make_async_copy(
            vals_hbm.at[pl.ds(base + g * CHUNK_ELEMS, CHUNK_ELEMS)],
            bufs[b],
            sems[b],
        )
        cp.start()
        return cp

    cps = [start_copy(g) for g in range(NBUF)]
    running = None
    for g in range(NCHUNK):
        cps[g].wait()
        m = _chunk_max(bufs[g % NBUF])
        running = m if running is None else jnp.maximum(running, m)
        if g + NBUF < NCHUNK:
            cps.append(start_copy(g + NBUF))
    outv[...] = running
    pltpu.sync_copy(outv, out_hbm.at[wid])


def _combine_kernel(parts_ref, o_ref):
    o_ref[0, 0] = jnp.max(parts_ref[...])


def kernel(values, prefix_sum):
    del prefix_sum
    parts = _sc_partial_max(values.reshape(-1))
    combined = pl.pallas_call(
        _combine_kernel,
        out_shape=jax.ShapeDtypeStruct((1, 1), jnp.float32),
        out_specs=pl.BlockSpec(memory_space=pltpu.SMEM),
    )(parts)
    return combined[0, 0]


# speedup vs baseline: 3.0093x; 2.4811x over previous
import functools

import jax
import jax.numpy as jnp
from jax import lax
from jax.experimental import pallas as pl
from jax.experimental.pallas import tpu as pltpu
from jax.experimental.pallas import tpu_sc as plsc

NC = 2
NS = 16
NW = NC * NS
L = 16

ROWS, COLS = 32768, 1024
RV = COLS // L
ROWS_PER_W = ROWS // NW
CHUNK_ROWS = 16
NCHUNK = ROWS_PER_W // CHUNK_ROWS
NBUF = 4
NGROUP = NCHUNK // NBUF
U = 8


def _chunk_max(buf, accs):
    @plsc.parallel_loop(0, CHUNK_ROWS, step=1, unroll=2, carry=tuple(accs))
    def folded(i, a):
        a = list(a)
        for j in range(RV):
            a[j % U] = jnp.maximum(a[j % U], buf[i, pl.ds(j * L, L)])
        return tuple(a)

    return list(folded)


_sc_mesh = plsc.VectorSubcoreMesh(core_axis_name="c", subcore_axis_name="s")


@functools.partial(
    pl.kernel,
    mesh=_sc_mesh,
    out_type=jax.ShapeDtypeStruct((NW, L), jnp.float32),
    scratch_types=[
        pltpu.VMEM((CHUNK_ROWS, COLS), jnp.float32),
        pltpu.VMEM((CHUNK_ROWS, COLS), jnp.float32),
        pltpu.VMEM((CHUNK_ROWS, COLS), jnp.float32),
        pltpu.VMEM((CHUNK_ROWS, COLS), jnp.float32),
        pltpu.VMEM((L,), jnp.float32),
        pltpu.SemaphoreType.DMA,
        pltpu.SemaphoreType.DMA,
        pltpu.SemaphoreType.DMA,
        pltpu.SemaphoreType.DMA,
    ],
)
def _sc_partial_max(vals, out_hbm, b0, b1, b2, b3, outv, s0, s1, s2, s3):
    wid = lax.axis_index("s") * NC + lax.axis_index("c")
    row0 = wid * ROWS_PER_W
    bufs = (b0, b1, b2, b3)
    sems = (s0, s1, s2, s3)

    def copy(g, b):
        return pltpu.make_async_copy(
            vals.at[pl.ds(row0 + g * CHUNK_ROWS, CHUNK_ROWS)], bufs[b], sems[b]
        )

    for b in range(NBUF):
        copy(b, b).start()

    neg_inf = jnp.full((L,), -jnp.inf, dtype=jnp.float32)

    def body(gg, accs):
        accs = list(accs)
        for b in range(NBUF):
            g = gg * NBUF + b
            copy(g, b).wait()
            accs = _chunk_max(bufs[b], accs)

            @pl.when(g + NBUF < NCHUNK)
            def _():
                copy(g + NBUF, b).start()

        return tuple(accs)

    accs = list(lax.fori_loop(0, NGROUP, body, (neg_inf,) * U))
    while len(accs) > 1:
        accs = [jnp.maximum(accs[i], accs[i + 1]) for i in range(0, len(accs), 2)]
    outv[...] = accs[0]
    pltpu.sync_copy(outv, out_hbm.at[wid])


def _combine_kernel(parts_ref, o_ref):
    o_ref[0, 0] = jnp.max(parts_ref[...])


def kernel(values, prefix_sum):
    del prefix_sum
    parts = _sc_partial_max(values)
    combined = pl.pallas_call(
        _combine_kernel,
        out_shape=jax.ShapeDtypeStruct((1, 1), jnp.float32),
        out_specs=pl.BlockSpec(memory_space=pltpu.SMEM),
    )(parts)
    return combined[0, 0]
